# Initial kernel scaffold; baseline (speedup 1.0000x reference)
import jax
import jax.numpy as jnp
from jax.experimental import pallas as pl


def kernel(h3_7, h3_8, s2_13, s2_14, T7, T8, T13, T14, W, b, gamma, beta):
    raise NotImplementedError("write your pallas kernel here")


# speedup vs baseline: 1.0279x; 1.0279x over previous
import functools

import jax
import jax.numpy as jnp
from jax import lax
from jax.experimental import pallas as pl
from jax.experimental.pallas import tpu as pltpu

B = 16384
DL = 32
D = 128


def _dense_body(c0, c1, c2, c3, w_ref, b_ref, g_ref, bt_ref, o_ref):
    x = b_ref[...]
    for k, c in enumerate((c0, c1, c2, c3)):
        x = x + lax.dot_general(
            c[...], w_ref[:, k * DL:(k + 1) * DL],
            (((1,), (1,)), ((), ())), preferred_element_type=jnp.float32)
    mu = jnp.mean(x, axis=-1, keepdims=True)
    xc = x - mu
    var = jnp.mean(xc * xc, axis=-1, keepdims=True)
    xn = xc * lax.rsqrt(var + 1e-5)
    o_ref[...] = xn * g_ref[...] + bt_ref[...]


def kernel(h3_7, h3_8, s2_13, s2_14, T7, T8, T13, T14, W, b, gamma, beta):
    e0 = jnp.take(T7, h3_7, axis=0)
    e1 = jnp.take(T8, h3_8, axis=0)
    e2 = jnp.take(T13, s2_13, axis=0)
    e3 = jnp.take(T14, s2_14, axis=0)
    blk = 1024
    out = pl.pallas_call(
        _dense_body,
        grid=(B // blk,),
        in_specs=[
            pl.BlockSpec((blk, DL), lambda i: (i, 0)),
            pl.BlockSpec((blk, DL), lambda i: (i, 0)),
            pl.BlockSpec((blk, DL), lambda i: (i, 0)),
            pl.BlockSpec((blk, DL), lambda i: (i, 0)),
            pl.BlockSpec((D, D), lambda i: (0, 0)),
            pl.BlockSpec((1, D), lambda i: (0, 0)),
            pl.BlockSpec((1, D), lambda i: (0, 0)),
            pl.BlockSpec((1, D), lambda i: (0, 0)),
        ],
        out_specs=pl.BlockSpec((blk, D), lambda i: (i, 0)),
        out_shape=jax.ShapeDtypeStruct((B, D), jnp.float32),
    )(e0, e1, e2, e3, W, b.reshape(1, D), gamma.reshape(1, D),
      beta.reshape(1, D))
    return out


# speedup vs baseline: 1.1316x; 1.1009x over previous
import functools

import jax
import jax.numpy as jnp
from jax import lax
from jax.experimental import pallas as pl
from jax.experimental.pallas import tpu as pltpu

B = 16384
DL = 32
D = 128


def _dense_body(c0, c1, c2, c3, w_ref, b_ref, g_ref, bt_ref, o_ref):
    x = b_ref[...]
    for k, c in enumerate((c0, c1, c2, c3)):
        x = x + lax.dot_general(
            c[...], w_ref[:, k * DL:(k + 1) * DL],
            (((0,), (1,)), ((), ())), preferred_element_type=jnp.float32)
    mu = jnp.mean(x, axis=-1, keepdims=True)
    xc = x - mu
    var = jnp.mean(xc * xc, axis=-1, keepdims=True)
    xn = xc * lax.rsqrt(var + 1e-5)
    o_ref[...] = xn * g_ref[...] + bt_ref[...]


def kernel(h3_7, h3_8, s2_13, s2_14, T7, T8, T13, T14, W, b, gamma, beta):
    e0 = jnp.take(T7.T, h3_7, axis=1)
    e1 = jnp.take(T8.T, h3_8, axis=1)
    e2 = jnp.take(T13.T, s2_13, axis=1)
    e3 = jnp.take(T14.T, s2_14, axis=1)
    blk = 1024
    out = pl.pallas_call(
        _dense_body,
        grid=(B // blk,),
        in_specs=[
            pl.BlockSpec((DL, blk), lambda i: (0, i)),
            pl.BlockSpec((DL, blk), lambda i: (0, i)),
            pl.BlockSpec((DL, blk), lambda i: (0, i)),
            pl.BlockSpec((DL, blk), lambda i: (0, i)),
            pl.BlockSpec((D, D), lambda i: (0, 0)),
            pl.BlockSpec((1, D), lambda i: (0, 0)),
            pl.BlockSpec((1, D), lambda i: (0, 0)),
            pl.BlockSpec((1, D), lambda i: (0, 0)),
        ],
        out_specs=pl.BlockSpec((blk, D), lambda i: (i, 0)),
        out_shape=jax.ShapeDtypeStruct((B, D), jnp.float32),
    )(e0, e1, e2, e3, W, b.reshape(1, D), gamma.reshape(1, D),
      beta.reshape(1, D))
    return out


# speedup vs baseline: 1.1699x; 1.0339x over previous
import functools

import jax
import jax.numpy as jnp
from jax import lax
from jax.experimental import pallas as pl
from jax.experimental.pallas import tpu as pltpu

B = 16384
DL = 32
D = 128


def _dense_body(c0, c1, c2, c3, w_ref, b_ref, g_ref, bt_ref, o_ref):
    x = b_ref[...]
    for k, c in enumerate((c0, c1, c2, c3)):
        x = x + lax.dot_general(
            c[...], w_ref[:, k * DL:(k + 1) * DL],
            (((0,), (1,)), ((), ())), preferred_element_type=jnp.float32)
    mu = jnp.mean(x, axis=-1, keepdims=True)
    xc = x - mu
    var = jnp.mean(xc * xc, axis=-1, keepdims=True)
    xn = xc * lax.rsqrt(var + 1e-5)
    o_ref[...] = xn * g_ref[...] + bt_ref[...]


def kernel(h3_7, h3_8, s2_13, s2_14, T7, T8, T13, T14, W, b, gamma, beta):
    e0 = T7.T.at[:, h3_7].get(mode="promise_in_bounds")
    e1 = T8.T.at[:, h3_8].get(mode="promise_in_bounds")
    e2 = T13.T.at[:, s2_13].get(mode="promise_in_bounds")
    e3 = T14.T.at[:, s2_14].get(mode="promise_in_bounds")
    blk = 1024
    out = pl.pallas_call(
        _dense_body,
        grid=(B // blk,),
        in_specs=[
            pl.BlockSpec((DL, blk), lambda i: (0, i)),
            pl.BlockSpec((DL, blk), lambda i: (0, i)),
            pl.BlockSpec((DL, blk), lambda i: (0, i)),
            pl.BlockSpec((DL, blk), lambda i: (0, i)),
            pl.BlockSpec((D, D), lambda i: (0, 0)),
            pl.BlockSpec((1, D), lambda i: (0, 0)),
            pl.BlockSpec((1, D), lambda i: (0, 0)),
            pl.BlockSpec((1, D), lambda i: (0, 0)),
        ],
        out_specs=pl.BlockSpec((blk, D), lambda i: (i, 0)),
        out_shape=jax.ShapeDtypeStruct((B, D), jnp.float32),
    )(e0, e1, e2, e3, W, b.reshape(1, D), gamma.reshape(1, D),
      beta.reshape(1, D))
    return out
